# Initial kernel scaffold; baseline (speedup 1.0000x reference)
import jax
import jax.numpy as jnp
from jax.experimental import pallas as pl


def kernel(query, keys, ref_point, Wq, bq, Wk, bk, Woff, boff, WA, bA, Wm, bm):
    raise NotImplementedError("write your pallas kernel here")


# speedup vs baseline: 1.2212x; 1.2212x over previous
import functools

import jax
import jax.numpy as jnp
from jax import lax
from jax.experimental import pallas as pl
from jax.experimental.pallas import tpu as pltpu
from jax.experimental.pallas import tpu_sc as plsc

NH = 8
NK = 4
NCORNER = 4
NJ = NK * NCORNER


def _proj_kernel(q_blk, k_blk, ref_blk, Wq, Wk, Woff, WA, bq, bk, boff, bA,
                 sf_out, idx_out, wgt_out):
    f32 = jnp.float32
    q = jnp.dot(q_blk[...], Wq[...], preferred_element_type=f32) + bq[...]
    sf_out[...] = jnp.dot(k_blk[...], Wk[...], preferred_element_type=f32) + bk[...]

    off = jnp.dot(q, Woff[...], preferred_element_type=f32) + boff[...]
    logits = jnp.dot(q, WA[...], preferred_element_type=f32) + bA[...]
    m = jnp.max(logits, axis=1, keepdims=True)
    e = jnp.exp(logits - m)
    gi = lax.broadcasted_iota(jnp.int32, (NH * NK, NH * NK), 0) // NK
    gj = lax.broadcasted_iota(jnp.int32, (NH * NK, NH * NK), 1) // NK
    G = (gi == gj).astype(f32)
    s = jnp.dot(e, G, preferred_element_type=f32)
    A = e / s

    offx = off[:, : NH * NK]
    offy = off[:, NH * NK:]
    rx = ref_blk[:, 0:1]
    ry = ref_blk[:, 1:2]
    px = rx * 31.0 + offx
    py = ry * 31.0 + offy
    ix = ((2.0 * px / 31.0) * 32.0 - 1.0) * 0.5
    iy = ((2.0 * py / 31.0) * 32.0 - 1.0) * 0.5
    ix0 = jnp.floor(ix)
    iy0 = jnp.floor(iy)
    wx1 = ix - ix0
    wx0 = 1.0 - wx1
    wy1 = iy - iy0
    wy0 = 1.0 - wy1
    vx0 = ((ix0 >= 0.0) & (ix0 <= 31.0)).astype(f32)
    vx1 = ((ix0 + 1.0 >= 0.0) & (ix0 + 1.0 <= 31.0)).astype(f32)
    vy0 = ((iy0 >= 0.0) & (iy0 <= 31.0)).astype(f32)
    vy1 = ((iy0 + 1.0 >= 0.0) & (iy0 + 1.0 <= 31.0)).astype(f32)
    x0 = jnp.clip(ix0, 0.0, 31.0).astype(jnp.int32)
    x1 = jnp.clip(ix0 + 1.0, 0.0, 31.0).astype(jnp.int32)
    y0 = jnp.clip(iy0, 0.0, 31.0).astype(jnp.int32)
    y1 = jnp.clip(iy0 + 1.0, 0.0, 31.0).astype(jnp.int32)

    idx_out[...] = jnp.concatenate(
        [y0 * 32 + x0, y0 * 32 + x1, y1 * 32 + x0, y1 * 32 + x1], axis=1)
    wgt_out[...] = jnp.concatenate(
        [A * wx0 * wy0 * vx0 * vy0, A * wx1 * wy0 * vx1 * vy0,
         A * wx0 * wy1 * vx0 * vy1, A * wx1 * wy1 * vx1 * vy1], axis=1)


def _run_proj(q2, k2, ref_rows, Wq, bq, Wk, bk, Woff_p, boff_p, WA, bA):
    R, C = q2.shape
    BR = 256
    grid = (R // BR,)
    row_spec = lambda n: pl.BlockSpec((BR, n), lambda i: (i, 0))
    full_spec = lambda a, b: pl.BlockSpec((a, b), lambda i: (0, 0))
    return pl.pallas_call(
        _proj_kernel,
        grid=grid,
        in_specs=[
            row_spec(C), row_spec(C), row_spec(2),
            full_spec(C, C), full_spec(C, C),
            full_spec(C, 2 * NH * NK), full_spec(C, NH * NK),
            full_spec(1, C), full_spec(1, C),
            full_spec(1, 2 * NH * NK), full_spec(1, NH * NK),
        ],
        out_specs=[row_spec(C), row_spec(NJ * NH), row_spec(NJ * NH)],
        out_shape=[
            jax.ShapeDtypeStruct((R, C), jnp.float32),
            jax.ShapeDtypeStruct((R, NJ * NH), jnp.int32),
            jax.ShapeDtypeStruct((R, NJ * NH), jnp.float32),
        ],
    )(q2, k2, ref_rows, Wq, Wk, Woff_p, WA,
      bq.reshape(1, -1), bk.reshape(1, -1), boff_p.reshape(1, -1),
      bA.reshape(1, -1))


def _matmul_kernel(x_blk, W, b, out):
    out[...] = jnp.dot(x_blk[...], W[...],
                       preferred_element_type=jnp.float32) + b[...]


def _run_matmul(x2, W, b):
    R, C = x2.shape
    BR = 256
    return pl.pallas_call(
        _matmul_kernel,
        grid=(R // BR,),
        in_specs=[
            pl.BlockSpec((BR, C), lambda i: (i, 0)),
            pl.BlockSpec((C, C), lambda i: (0, 0)),
            pl.BlockSpec((1, C), lambda i: (0, 0)),
        ],
        out_specs=pl.BlockSpec((BR, C), lambda i: (i, 0)),
        out_shape=jax.ShapeDtypeStruct((R, C), jnp.float32),
    )(x2, W, b.reshape(1, -1))


def _make_sample_kernel(BNH, HW, DK, interpret=False):
    HALF = HW // 2
    CHUNK = 128
    NPG = 8
    mesh = plsc.VectorSubcoreMesh(core_axis_name="c", subcore_axis_name="s",
                                  num_cores=2, num_subcores=16)

    @functools.partial(
        pl.kernel,
        mesh=mesh,
        out_type=jax.ShapeDtypeStruct((BNH, DK, HW), jnp.float32),
        scratch_types=[
            pltpu.VMEM((DK, HW), jnp.float32),
            pltpu.VMEM((NJ, HALF), jnp.int32),
            pltpu.VMEM((NJ, HALF), jnp.float32),
            pltpu.VMEM((DK, CHUNK), jnp.float32),
        ],
        compiler_params=pltpu.CompilerParams(use_tc_tiling_on_sc=False,
                                             needs_layout_passes=False),
        interpret=interpret,
    )
    def sample(sf_hbm, idx_hbm, wgt_hbm, out_hbm, table_v, idx_v, wgt_v, out_v):
        bh = lax.axis_index("s")
        half = lax.axis_index("c")
        pbase = half * HALF
        pltpu.sync_copy(sf_hbm.at[bh], table_v)
        pltpu.sync_copy(idx_hbm.at[bh, :, pl.ds(pbase, HALF)], idx_v)
        pltpu.sync_copy(wgt_hbm.at[bh, :, pl.ds(pbase, HALF)], wgt_v)

        def chunk_body(ci, _):
            def pg_body(pg, _):
                p0 = ci * CHUNK + pg * 16
                ivs = [idx_v[j, pl.ds(p0, 16)] for j in range(NJ)]
                wvs = [wgt_v[j, pl.ds(p0, 16)] for j in range(NJ)]
                for d in range(DK):
                    dvec = jnp.full((16,), d, jnp.int32)
                    terms = [wvs[j] * plsc.load_gather(table_v, [dvec, ivs[j]])
                             for j in range(NJ)]
                    while len(terms) > 1:
                        terms = [terms[t] + terms[t + 1]
                                 for t in range(0, len(terms) - 1, 2)] + \
                                (terms[-1:] if len(terms) % 2 else [])
                    out_v[d, pl.ds(pg * 16, 16)] = terms[0]
                return 0
            lax.fori_loop(0, NPG, pg_body, 0)
            pltpu.sync_copy(out_v,
                            out_hbm.at[bh, :, pl.ds(pbase + ci * CHUNK, CHUNK)])
            return 0

        lax.fori_loop(0, HALF // CHUNK, chunk_body, 0)

    return sample


def kernel(query, keys, ref_point, Wq, bq, Wk, bk, Woff, boff, WA, bA, Wm, bm):
    B, H, W, C = query.shape
    HW = H * W
    DK = C // NH
    f32 = jnp.float32

    q2 = query.reshape(B * HW, C)
    k2 = keys.reshape(B * HW, C)
    ref_rows = jnp.tile(ref_point.reshape(HW, 2), (B, 1))

    cols = jnp.arange(NH * NK)
    perm = jnp.concatenate([(cols // NK) * (NK * 2) + (cols % NK) * 2,
                            (cols // NK) * (NK * 2) + (cols % NK) * 2 + 1])
    Woff_p = Woff[:, perm]
    boff_p = boff[perm]

    sf, idx, wgt = _run_proj(q2, k2, ref_rows, Wq, bq, Wk, bk,
                             Woff_p, boff_p, WA, bA)

    idx_sc = (idx.reshape(B, HW, NCORNER, NH, NK)
              .transpose(0, 3, 2, 4, 1).reshape(B * NH, NJ, HW))
    wgt_sc = (wgt.reshape(B, HW, NCORNER, NH, NK)
              .transpose(0, 3, 2, 4, 1).reshape(B * NH, NJ, HW))

    sf_sc = (sf.reshape(B, HW, NH, DK).transpose(0, 2, 3, 1)
             .reshape(B * NH, DK, HW))

    feat = _make_sample_kernel(B * NH, HW, DK)(sf_sc, idx_sc, wgt_sc)

    out = _run_matmul(feat.transpose(0, 2, 1).reshape(B * HW, C), Wm, bm)
    return out.reshape(B, H, W, C)


# speedup vs baseline: 2.0914x; 1.7126x over previous
import functools

import jax
import jax.numpy as jnp
from jax import lax
from jax.experimental import pallas as pl
from jax.experimental.pallas import tpu as pltpu
from jax.experimental.pallas import tpu_sc as plsc

NH = 8
NK = 4
NCORNER = 4
NJ = NK * NCORNER


def _proj_kernel(dk, q_blk, k_blk, ref_blk, Wq, Wk, Woff, WA, bq, bk, boff, bA,
                 sf_out, idx_out, wgt_out):
    f32 = jnp.float32
    q = jnp.dot(q_blk[...], Wq[...], preferred_element_type=f32) + bq[...]
    sf_out[...] = jnp.dot(k_blk[...], Wk[...], preferred_element_type=f32) + bk[...]

    off = jnp.dot(q, Woff[...], preferred_element_type=f32) + boff[...]
    logits = jnp.dot(q, WA[...], preferred_element_type=f32) + bA[...]
    m = jnp.max(logits, axis=1, keepdims=True)
    e = jnp.exp(logits - m)
    gi = lax.broadcasted_iota(jnp.int32, (NH * NK, NH * NK), 0) // NK
    gj = lax.broadcasted_iota(jnp.int32, (NH * NK, NH * NK), 1) // NK
    G = (gi == gj).astype(f32)
    s = jnp.dot(e, G, preferred_element_type=f32)
    A = e / s

    offx = off[:, : NH * NK]
    offy = off[:, NH * NK:]
    rx = ref_blk[:, 0:1]
    ry = ref_blk[:, 1:2]
    px = rx * 31.0 + offx
    py = ry * 31.0 + offy
    ix = ((2.0 * px / 31.0) * 32.0 - 1.0) * 0.5
    iy = ((2.0 * py / 31.0) * 32.0 - 1.0) * 0.5
    ix0 = jnp.floor(ix)
    iy0 = jnp.floor(iy)
    wx1 = ix - ix0
    wx0 = 1.0 - wx1
    wy1 = iy - iy0
    wy0 = 1.0 - wy1
    vx0 = ((ix0 >= 0.0) & (ix0 <= 31.0)).astype(f32)
    vx1 = ((ix0 + 1.0 >= 0.0) & (ix0 + 1.0 <= 31.0)).astype(f32)
    vy0 = ((iy0 >= 0.0) & (iy0 <= 31.0)).astype(f32)
    vy1 = ((iy0 + 1.0 >= 0.0) & (iy0 + 1.0 <= 31.0)).astype(f32)
    x0 = jnp.clip(ix0, 0.0, 31.0).astype(jnp.int32)
    x1 = jnp.clip(ix0 + 1.0, 0.0, 31.0).astype(jnp.int32)
    y0 = jnp.clip(iy0, 0.0, 31.0).astype(jnp.int32)
    y1 = jnp.clip(iy0 + 1.0, 0.0, 31.0).astype(jnp.int32)

    idx_out[...] = jnp.concatenate(
        [y0 * 32 + x0, y0 * 32 + x1, y1 * 32 + x0, y1 * 32 + x1], axis=1) * dk
    wgt_out[...] = jnp.concatenate(
        [A * wx0 * wy0 * vx0 * vy0, A * wx1 * wy0 * vx1 * vy0,
         A * wx0 * wy1 * vx0 * vy1, A * wx1 * wy1 * vx1 * vy1], axis=1)


def _run_proj(q2, k2, ref_rows, Wq, bq, Wk, bk, Woff_p, boff_p, WA, bA):
    R, C = q2.shape
    BR = 256
    grid = (R // BR,)
    row_spec = lambda n: pl.BlockSpec((BR, n), lambda i: (i, 0))
    full_spec = lambda a, b: pl.BlockSpec((a, b), lambda i: (0, 0))
    return pl.pallas_call(
        functools.partial(_proj_kernel, C // NH),
        grid=grid,
        in_specs=[
            row_spec(C), row_spec(C), row_spec(2),
            full_spec(C, C), full_spec(C, C),
            full_spec(C, 2 * NH * NK), full_spec(C, NH * NK),
            full_spec(1, C), full_spec(1, C),
            full_spec(1, 2 * NH * NK), full_spec(1, NH * NK),
        ],
        out_specs=[row_spec(C), row_spec(NJ * NH), row_spec(NJ * NH)],
        out_shape=[
            jax.ShapeDtypeStruct((R, C), jnp.float32),
            jax.ShapeDtypeStruct((R, NJ * NH), jnp.int32),
            jax.ShapeDtypeStruct((R, NJ * NH), jnp.float32),
        ],
    )(q2, k2, ref_rows, Wq, Wk, Woff_p, WA,
      bq.reshape(1, -1), bk.reshape(1, -1), boff_p.reshape(1, -1),
      bA.reshape(1, -1))


def _matmul_kernel(x_blk, W, b, out):
    out[...] = jnp.dot(x_blk[...], W[...],
                       preferred_element_type=jnp.float32) + b[...]


def _run_matmul(x2, W, b):
    R, C = x2.shape
    BR = 256
    return pl.pallas_call(
        _matmul_kernel,
        grid=(R // BR,),
        in_specs=[
            pl.BlockSpec((BR, C), lambda i: (i, 0)),
            pl.BlockSpec((C, C), lambda i: (0, 0)),
            pl.BlockSpec((1, C), lambda i: (0, 0)),
        ],
        out_specs=pl.BlockSpec((BR, C), lambda i: (i, 0)),
        out_shape=jax.ShapeDtypeStruct((R, C), jnp.float32),
    )(x2, W, b.reshape(1, -1))


def _make_sample_kernel(BNH, HW, DK, interpret=False):
    HALF = HW // 2
    CHUNK = 128
    NSEG = DK // 16
    mesh = plsc.VectorSubcoreMesh(core_axis_name="c", subcore_axis_name="s",
                                  num_cores=2, num_subcores=16)

    @functools.partial(
        pl.kernel,
        mesh=mesh,
        out_type=jax.ShapeDtypeStruct((BNH, HW, DK), jnp.float32),
        scratch_types=[
            pltpu.VMEM((HW * DK,), jnp.float32),
            pltpu.VMEM((HALF, NJ), jnp.int32),
            pltpu.VMEM((HALF, NJ), jnp.float32),
            pltpu.VMEM((CHUNK, DK), jnp.float32),
        ],
        compiler_params=pltpu.CompilerParams(use_tc_tiling_on_sc=False,
                                             needs_layout_passes=False),
        interpret=interpret,
    )
    def sample(sf_hbm, idx_hbm, wgt_hbm, out_hbm, table_v, idx_v, wgt_v, out_v):
        bh = lax.axis_index("s")
        half = lax.axis_index("c")
        pbase = half * HALF
        pltpu.sync_copy(sf_hbm.at[bh], table_v)
        pltpu.sync_copy(idx_hbm.at[bh, pl.ds(pbase, HALF), :], idx_v)
        pltpu.sync_copy(wgt_hbm.at[bh, pl.ds(pbase, HALF), :], wgt_v)

        def pos_body(p, _):
            iv = idx_v[p, :]
            wv = wgt_v[p, :]
            prow = lax.rem(p, CHUNK)
            accs = None
            for j in range(NJ):
                row = iv[j]
                w = jnp.full((16,), wv[j])
                segs = [w * table_v[pl.ds(row + 16 * c, 16)]
                        for c in range(NSEG)]
                accs = segs if accs is None else \
                    [a + s for a, s in zip(accs, segs)]
            for c in range(NSEG):
                out_v[prow, pl.ds(16 * c, 16)] = accs[c]

            @pl.when(prow == CHUNK - 1)
            def _():
                pltpu.sync_copy(
                    out_v,
                    out_hbm.at[bh, pl.ds(pbase + (p - (CHUNK - 1)), CHUNK), :])
            return 0

        lax.fori_loop(0, HALF, pos_body, 0)

    return sample


def kernel(query, keys, ref_point, Wq, bq, Wk, bk, Woff, boff, WA, bA, Wm, bm):
    B, H, W, C = query.shape
    HW = H * W
    DK = C // NH
    f32 = jnp.float32

    q2 = query.reshape(B * HW, C)
    k2 = keys.reshape(B * HW, C)
    ref_rows = jnp.tile(ref_point.reshape(HW, 2), (B, 1))

    cols = jnp.arange(NH * NK)
    perm = jnp.concatenate([(cols // NK) * (NK * 2) + (cols % NK) * 2,
                            (cols // NK) * (NK * 2) + (cols % NK) * 2 + 1])
    Woff_p = Woff[:, perm]
    boff_p = boff[perm]

    sf, idx, wgt = _run_proj(q2, k2, ref_rows, Wq, bq, Wk, bk,
                             Woff_p, boff_p, WA, bA)

    idx_sc = (idx.reshape(B, HW, NCORNER, NH, NK)
              .transpose(0, 3, 1, 2, 4).reshape(B * NH, HW, NJ))
    wgt_sc = (wgt.reshape(B, HW, NCORNER, NH, NK)
              .transpose(0, 3, 1, 2, 4).reshape(B * NH, HW, NJ))

    sf_sc = (sf.reshape(B, HW, NH, DK).transpose(0, 2, 1, 3)
             .reshape(B * NH, HW * DK))

    feat = _make_sample_kernel(B * NH, HW, DK)(sf_sc, idx_sc, wgt_sc)

    out = _run_matmul(feat.reshape(B * HW, C), Wm, bm)
    return out.reshape(B, H, W, C)


# speedup vs baseline: 2.6008x; 1.2436x over previous
import functools

import jax
import jax.numpy as jnp
from jax import lax
from jax.experimental import pallas as pl
from jax.experimental.pallas import tpu as pltpu
from jax.experimental.pallas import tpu_sc as plsc

NH = 8
NK = 4
NCORNER = 4
NJ = NK * NCORNER


def _proj_kernel(dk, q_blk, k_blk, ref_blk, Wq, Wk, Woff, WA, bq, bk, boff, bA,
                 sf_out, idx_out, wgt_out):
    f32 = jnp.float32
    q = jnp.dot(q_blk[...], Wq[...], preferred_element_type=f32) + bq[...]
    sf_out[...] = jnp.dot(k_blk[...], Wk[...], preferred_element_type=f32) + bk[...]

    off = jnp.dot(q, Woff[...], preferred_element_type=f32) + boff[...]
    logits = jnp.dot(q, WA[...], preferred_element_type=f32) + bA[...]
    m = jnp.max(logits, axis=1, keepdims=True)
    e = jnp.exp(logits - m)
    gi = lax.broadcasted_iota(jnp.int32, (NH * NK, NH * NK), 0) // NK
    gj = lax.broadcasted_iota(jnp.int32, (NH * NK, NH * NK), 1) // NK
    G = (gi == gj).astype(f32)
    s = jnp.dot(e, G, preferred_element_type=f32)
    A = e / s

    offx = off[:, : NH * NK]
    offy = off[:, NH * NK:]
    rx = ref_blk[:, 0:1]
    ry = ref_blk[:, 1:2]
    px = rx * 31.0 + offx
    py = ry * 31.0 + offy
    ix = ((2.0 * px / 31.0) * 32.0 - 1.0) * 0.5
    iy = ((2.0 * py / 31.0) * 32.0 - 1.0) * 0.5
    ix0 = jnp.floor(ix)
    iy0 = jnp.floor(iy)
    wx1 = ix - ix0
    wx0 = 1.0 - wx1
    wy1 = iy - iy0
    wy0 = 1.0 - wy1
    vx0 = ((ix0 >= 0.0) & (ix0 <= 31.0)).astype(f32)
    vx1 = ((ix0 + 1.0 >= 0.0) & (ix0 + 1.0 <= 31.0)).astype(f32)
    vy0 = ((iy0 >= 0.0) & (iy0 <= 31.0)).astype(f32)
    vy1 = ((iy0 + 1.0 >= 0.0) & (iy0 + 1.0 <= 31.0)).astype(f32)
    x0 = jnp.clip(ix0, 0.0, 31.0).astype(jnp.int32)
    x1 = jnp.clip(ix0 + 1.0, 0.0, 31.0).astype(jnp.int32)
    y0 = jnp.clip(iy0, 0.0, 31.0).astype(jnp.int32)
    y1 = jnp.clip(iy0 + 1.0, 0.0, 31.0).astype(jnp.int32)

    idx_c = [y0 * 32 + x0, y0 * 32 + x1, y1 * 32 + x0, y1 * 32 + x1]
    wgt_c = [A * wx0 * wy0 * vx0 * vy0, A * wx1 * wy0 * vx1 * vy0,
             A * wx0 * wy1 * vx0 * vy1, A * wx1 * wy1 * vx1 * vy1]
    idx_out[...] = jnp.concatenate(
        [a[:, nh * NK:(nh + 1) * NK] for nh in range(NH) for a in idx_c],
        axis=1)
    wgt_out[...] = jnp.concatenate(
        [a[:, nh * NK:(nh + 1) * NK] for nh in range(NH) for a in wgt_c],
        axis=1)


def _run_proj(q2, k2, ref_rows, Wq, bq, Wk, bk, Woff_p, boff_p, WA, bA):
    R, C = q2.shape
    BR = 256
    grid = (R // BR,)
    nrb = ref_rows.shape[0] // BR
    row_spec = lambda n: pl.BlockSpec((BR, n), lambda i: (i, 0))
    full_spec = lambda a, b: pl.BlockSpec((a, b), lambda i: (0, 0))
    return pl.pallas_call(
        functools.partial(_proj_kernel, C // NH),
        grid=grid,
        in_specs=[
            row_spec(C), row_spec(C),
            pl.BlockSpec((BR, 2), lambda i: (i % nrb, 0)),
            full_spec(C, C), full_spec(C, C),
            full_spec(C, 2 * NH * NK), full_spec(C, NH * NK),
            full_spec(1, C), full_spec(1, C),
            full_spec(1, 2 * NH * NK), full_spec(1, NH * NK),
        ],
        out_specs=[row_spec(C), row_spec(NJ * NH), row_spec(NJ * NH)],
        out_shape=[
            jax.ShapeDtypeStruct((R, C), jnp.float32),
            jax.ShapeDtypeStruct((R, NJ * NH), jnp.int32),
            jax.ShapeDtypeStruct((R, NJ * NH), jnp.float32),
        ],
    )(q2, k2, ref_rows, Wq, Wk, Woff_p, WA,
      bq.reshape(1, -1), bk.reshape(1, -1), boff_p.reshape(1, -1),
      bA.reshape(1, -1))


def _matmul_kernel(x_blk, W, b, out):
    out[...] = jnp.dot(x_blk[...], W[...],
                       preferred_element_type=jnp.float32) + b[...]


def _run_matmul(x2, W, b):
    R, C = x2.shape
    BR = 256
    return pl.pallas_call(
        _matmul_kernel,
        grid=(R // BR,),
        in_specs=[
            pl.BlockSpec((BR, C), lambda i: (i, 0)),
            pl.BlockSpec((C, C), lambda i: (0, 0)),
            pl.BlockSpec((1, C), lambda i: (0, 0)),
        ],
        out_specs=pl.BlockSpec((BR, C), lambda i: (i, 0)),
        out_shape=jax.ShapeDtypeStruct((R, C), jnp.float32),
    )(x2, W, b.reshape(1, -1))


def _make_sample_kernel(BNH, HW, DK, interpret=False):
    HALF = HW // 2
    CHUNK = 128
    NSEG = DK // 16
    mesh = plsc.VectorSubcoreMesh(core_axis_name="c", subcore_axis_name="s",
                                  num_cores=2, num_subcores=16)

    @functools.partial(
        pl.kernel,
        mesh=mesh,
        out_type=jax.ShapeDtypeStruct((BNH, HW, DK), jnp.float32),
        scratch_types=[
            pltpu.VMEM((HW, DK), jnp.float32),
            pltpu.VMEM((HALF, NJ), jnp.int32),
            pltpu.VMEM((HALF, NJ), jnp.float32),
            pltpu.VMEM((CHUNK, DK), jnp.float32),
            pltpu.SemaphoreType.DMA,
        ],
        compiler_params=pltpu.CompilerParams(use_tc_tiling_on_sc=False,
                                             needs_layout_passes=False),
        interpret=interpret,
    )
    def sample(sf_hbm, idx_hbm, wgt_hbm, out_hbm, table_v, idx_v, wgt_v,
               out_v, sem):
        bh = lax.axis_index("s")
        half = lax.axis_index("c")
        b = bh // NH
        nh = bh % NH
        pbase = half * HALF
        rbase = b * HW + pbase
        copies = [
            pltpu.async_copy(
                sf_hbm.at[pl.ds(b * HW, HW), pl.ds(nh * DK, DK)], table_v,
                sem),
            pltpu.async_copy(
                idx_hbm.at[pl.ds(rbase, HALF), pl.ds(nh * NJ, NJ)], idx_v,
                sem),
            pltpu.async_copy(
                wgt_hbm.at[pl.ds(rbase, HALF), pl.ds(nh * NJ, NJ)], wgt_v,
                sem),
        ]
        for cp in copies:
            cp.wait()

        def pos_body(p, _):
            iv = idx_v[p, :]
            wv = wgt_v[p, :]
            prow = lax.rem(p, CHUNK)
            accs = None
            for j in range(NJ):
                row = iv[j]
                w = jnp.full((16,), wv[j])
                segs = [w * table_v[row, pl.ds(16 * c, 16)]
                        for c in range(NSEG)]
                accs = segs if accs is None else \
                    [a + s for a, s in zip(accs, segs)]
            for c in range(NSEG):
                out_v[prow, pl.ds(16 * c, 16)] = accs[c]

            @pl.when(prow == CHUNK - 1)
            def _():
                pltpu.sync_copy(
                    out_v,
                    out_hbm.at[bh, pl.ds(pbase + (p - (CHUNK - 1)), CHUNK), :])
            return 0

        lax.fori_loop(0, HALF, pos_body, 0)

    return sample


def kernel(query, keys, ref_point, Wq, bq, Wk, bk, Woff, boff, WA, bA, Wm, bm):
    B, H, W, C = query.shape
    HW = H * W
    DK = C // NH
    f32 = jnp.float32

    q2 = query.reshape(B * HW, C)
    k2 = keys.reshape(B * HW, C)
    ref_rows = ref_point.reshape(HW, 2)

    cols = jnp.arange(NH * NK)
    perm = jnp.concatenate([(cols // NK) * (NK * 2) + (cols % NK) * 2,
                            (cols // NK) * (NK * 2) + (cols % NK) * 2 + 1])
    Woff_p = Woff[:, perm]
    boff_p = boff[perm]

    sf, idx, wgt = _run_proj(q2, k2, ref_rows, Wq, bq, Wk, bk,
                             Woff_p, boff_p, WA, bA)

    feat = _make_sample_kernel(B * NH, HW, DK)(sf, idx, wgt)

    out = _run_matmul(feat.reshape(B * HW, C), Wm, bm)
    return out.reshape(B, H, W, C)


# speedup vs baseline: 2.6984x; 1.0375x over previous
import functools

import jax
import jax.numpy as jnp
from jax import lax
from jax.experimental import pallas as pl
from jax.experimental.pallas import tpu as pltpu
from jax.experimental.pallas import tpu_sc as plsc

NH = 8
NK = 4
NCORNER = 4
NJ = NK * NCORNER


def _proj_kernel(dk, q_blk, k_blk, ref_blk, Wq, Wk, Woff, WA, bq, bk, boff, bA,
                 sf_out, idx_out, wgt_out):
    f32 = jnp.float32
    q = jnp.dot(q_blk[...], Wq[...], preferred_element_type=f32) + bq[...]
    sf_out[...] = jnp.dot(k_blk[...], Wk[...], preferred_element_type=f32) + bk[...]

    off = jnp.dot(q, Woff[...], preferred_element_type=f32) + boff[...]
    logits = jnp.dot(q, WA[...], preferred_element_type=f32) + bA[...]
    m = jnp.max(logits, axis=1, keepdims=True)
    e = jnp.exp(logits - m)
    gi = lax.broadcasted_iota(jnp.int32, (NH * NK, NH * NK), 0) // NK
    gj = lax.broadcasted_iota(jnp.int32, (NH * NK, NH * NK), 1) // NK
    G = (gi == gj).astype(f32)
    s = jnp.dot(e, G, preferred_element_type=f32)
    A = e / s

    offx = off[:, : NH * NK]
    offy = off[:, NH * NK:]
    rx = ref_blk[:, 0:1]
    ry = ref_blk[:, 1:2]
    px = rx * 31.0 + offx
    py = ry * 31.0 + offy
    ix = ((2.0 * px / 31.0) * 32.0 - 1.0) * 0.5
    iy = ((2.0 * py / 31.0) * 32.0 - 1.0) * 0.5
    ix0 = jnp.floor(ix)
    iy0 = jnp.floor(iy)
    wx1 = ix - ix0
    wx0 = 1.0 - wx1
    wy1 = iy - iy0
    wy0 = 1.0 - wy1
    vx0 = ((ix0 >= 0.0) & (ix0 <= 31.0)).astype(f32)
    vx1 = ((ix0 + 1.0 >= 0.0) & (ix0 + 1.0 <= 31.0)).astype(f32)
    vy0 = ((iy0 >= 0.0) & (iy0 <= 31.0)).astype(f32)
    vy1 = ((iy0 + 1.0 >= 0.0) & (iy0 + 1.0 <= 31.0)).astype(f32)
    x0 = jnp.clip(ix0, 0.0, 31.0).astype(jnp.int32)
    x1 = jnp.clip(ix0 + 1.0, 0.0, 31.0).astype(jnp.int32)
    y0 = jnp.clip(iy0, 0.0, 31.0).astype(jnp.int32)
    y1 = jnp.clip(iy0 + 1.0, 0.0, 31.0).astype(jnp.int32)

    idx_c = [y0 * 32 + x0, y0 * 32 + x1, y1 * 32 + x0, y1 * 32 + x1]
    wgt_c = [A * wx0 * wy0 * vx0 * vy0, A * wx1 * wy0 * vx1 * vy0,
             A * wx0 * wy1 * vx0 * vy1, A * wx1 * wy1 * vx1 * vy1]
    idx_out[...] = jnp.concatenate(
        [a[:, nh * NK:(nh + 1) * NK] for nh in range(NH) for a in idx_c],
        axis=1)
    wgt_out[...] = jnp.concatenate(
        [a[:, nh * NK:(nh + 1) * NK] for nh in range(NH) for a in wgt_c],
        axis=1)


def _run_proj(q2, k2, ref_rows, Wq, bq, Wk, bk, Woff_p, boff_p, WA, bA):
    R, C = q2.shape
    BR = 256
    grid = (R // BR,)
    nrb = ref_rows.shape[0] // BR
    row_spec = lambda n: pl.BlockSpec((BR, n), lambda i: (i, 0))
    full_spec = lambda a, b: pl.BlockSpec((a, b), lambda i: (0, 0))
    return pl.pallas_call(
        functools.partial(_proj_kernel, C // NH),
        grid=grid,
        in_specs=[
            row_spec(C), row_spec(C),
            pl.BlockSpec((BR, 2), lambda i: (i % nrb, 0)),
            full_spec(C, C), full_spec(C, C),
            full_spec(C, 2 * NH * NK), full_spec(C, NH * NK),
            full_spec(1, C), full_spec(1, C),
            full_spec(1, 2 * NH * NK), full_spec(1, NH * NK),
        ],
        out_specs=[row_spec(C), row_spec(NJ * NH), row_spec(NJ * NH)],
        out_shape=[
            jax.ShapeDtypeStruct((R, C), jnp.float32),
            jax.ShapeDtypeStruct((R, NJ * NH), jnp.int32),
            jax.ShapeDtypeStruct((R, NJ * NH), jnp.float32),
        ],
    )(q2, k2, ref_rows, Wq, Wk, Woff_p, WA,
      bq.reshape(1, -1), bk.reshape(1, -1), boff_p.reshape(1, -1),
      bA.reshape(1, -1))


def _matmul_kernel(x_blk, W, b, out):
    out[...] = jnp.dot(x_blk[...], W[...],
                       preferred_element_type=jnp.float32) + b[...]


def _run_matmul(x2, W, b):
    R, C = x2.shape
    BR = 256
    return pl.pallas_call(
        _matmul_kernel,
        grid=(R // BR,),
        in_specs=[
            pl.BlockSpec((BR, C), lambda i: (i, 0)),
            pl.BlockSpec((C, C), lambda i: (0, 0)),
            pl.BlockSpec((1, C), lambda i: (0, 0)),
        ],
        out_specs=pl.BlockSpec((BR, C), lambda i: (i, 0)),
        out_shape=jax.ShapeDtypeStruct((R, C), jnp.float32),
    )(x2, W, b.reshape(1, -1))


def _make_sample_kernel(BNH, HW, DK, interpret=False):
    HALF = HW // 2
    CHUNK = 64
    NSEG = DK // 16
    mesh = plsc.VectorSubcoreMesh(core_axis_name="c", subcore_axis_name="s",
                                  num_cores=2, num_subcores=16)

    @functools.partial(
        pl.kernel,
        mesh=mesh,
        out_type=jax.ShapeDtypeStruct((BNH, HW, DK), jnp.float32),
        scratch_types=[
            pltpu.VMEM((HW, DK), jnp.float32),
            pltpu.VMEM((HALF, NJ), jnp.int32),
            pltpu.VMEM((HALF, NJ), jnp.float32),
            pltpu.VMEM((2, CHUNK, DK), jnp.float32),
            pltpu.SemaphoreType.DMA,
            pltpu.SemaphoreType.DMA,
        ],
        compiler_params=pltpu.CompilerParams(use_tc_tiling_on_sc=False,
                                             needs_layout_passes=False),
        interpret=interpret,
    )
    def sample(sf_hbm, idx_hbm, wgt_hbm, out_hbm, table_v, idx_v, wgt_v,
               out_v, sem, osem):
        bh = lax.axis_index("s")
        half = lax.axis_index("c")
        b = bh // NH
        nh = bh % NH
        pbase = half * HALF
        rbase = b * HW + pbase
        copies = [
            pltpu.async_copy(
                sf_hbm.at[pl.ds(b * HW, HW), pl.ds(nh * DK, DK)], table_v,
                sem),
            pltpu.async_copy(
                idx_hbm.at[pl.ds(rbase, HALF), pl.ds(nh * NJ, NJ)], idx_v,
                sem),
            pltpu.async_copy(
                wgt_hbm.at[pl.ds(rbase, HALF), pl.ds(nh * NJ, NJ)], wgt_v,
                sem),
        ]
        for cp in copies:
            cp.wait()

        NCH = HALF // CHUNK

        def one_pos(p, buf, prow):
            iv = idx_v[p, :]
            wv = wgt_v[p, :]
            accs = None
            for j in range(NJ):
                row = iv[j]
                w = jnp.full((16,), wv[j])
                segs = [w * table_v[row, pl.ds(16 * c, 16)]
                        for c in range(NSEG)]
                accs = segs if accs is None else \
                    [a + s for a, s in zip(accs, segs)]
            for c in range(NSEG):
                out_v[buf, prow, pl.ds(16 * c, 16)] = accs[c]

        def chunk_body(ci, _):
            buf = lax.rem(ci, 2)

            @pl.when(ci >= 2)
            def _():
                pltpu.make_async_copy(
                    out_v.at[buf], out_hbm.at[bh, pl.ds(pbase, CHUNK), :],
                    osem).wait()

            def pair_body(q, _):
                prow = q * 2
                one_pos(ci * CHUNK + prow, buf, prow)
                one_pos(ci * CHUNK + prow + 1, buf, prow + 1)
                return 0

            lax.fori_loop(0, CHUNK // 2, pair_body, 0)
            pltpu.async_copy(
                out_v.at[buf],
                out_hbm.at[bh, pl.ds(pbase + ci * CHUNK, CHUNK), :], osem)
            return 0

        lax.fori_loop(0, NCH, chunk_body, 0)
        for t in range(min(2, NCH)):
            pltpu.make_async_copy(
                out_v.at[t], out_hbm.at[bh, pl.ds(pbase, CHUNK), :],
                osem).wait()

    return sample


def kernel(query, keys, ref_point, Wq, bq, Wk, bk, Woff, boff, WA, bA, Wm, bm):
    B, H, W, C = query.shape
    HW = H * W
    DK = C // NH
    f32 = jnp.float32

    q2 = query.reshape(B * HW, C)
    k2 = keys.reshape(B * HW, C)
    ref_rows = ref_point.reshape(HW, 2)

    cols = jnp.arange(NH * NK)
    perm = jnp.concatenate([(cols // NK) * (NK * 2) + (cols % NK) * 2,
                            (cols // NK) * (NK * 2) + (cols % NK) * 2 + 1])
    Woff_p = Woff[:, perm]
    boff_p = boff[perm]

    sf, idx, wgt = _run_proj(q2, k2, ref_rows, Wq, bq, Wk, bk,
                             Woff_p, boff_p, WA, bA)

    feat = _make_sample_kernel(B * NH, HW, DK)(sf, idx, wgt)

    out = _run_matmul(feat.reshape(B * HW, C), Wm, bm)
    return out.reshape(B, H, W, C)


# speedup vs baseline: 2.9321x; 1.0866x over previous
import functools

import jax
import jax.numpy as jnp
from jax import lax
from jax.experimental import pallas as pl
from jax.experimental.pallas import tpu as pltpu
from jax.experimental.pallas import tpu_sc as plsc

NH = 8
NK = 4
NCORNER = 4
NJ = NK * NCORNER


def _proj_kernel(dk, q_blk, k_blk, ref_blk, Wq, Wk, Woff, WA, bq, bk, boff, bA,
                 sf_out, idx_out, wgt_out):
    f32 = jnp.float32
    q = jnp.dot(q_blk[...], Wq[...], preferred_element_type=f32) + bq[...]
    sf = jnp.dot(k_blk[...], Wk[...], preferred_element_type=f32) + bk[...]
    sf_out[...] = sf.reshape(sf_out.shape)

    off = jnp.dot(q, Woff[...], preferred_element_type=f32) + boff[...]
    logits = jnp.dot(q, WA[...], preferred_element_type=f32) + bA[...]
    m = jnp.max(logits, axis=1, keepdims=True)
    e = jnp.exp(logits - m)
    gi = lax.broadcasted_iota(jnp.int32, (NH * NK, NH * NK), 0) // NK
    gj = lax.broadcasted_iota(jnp.int32, (NH * NK, NH * NK), 1) // NK
    G = (gi == gj).astype(f32)
    s = jnp.dot(e, G, preferred_element_type=f32)
    A = e / s

    offx = off[:, : NH * NK]
    offy = off[:, NH * NK:]
    rx = ref_blk[:, 0:1]
    ry = ref_blk[:, 1:2]
    px = rx * 31.0 + offx
    py = ry * 31.0 + offy
    ix = ((2.0 * px / 31.0) * 32.0 - 1.0) * 0.5
    iy = ((2.0 * py / 31.0) * 32.0 - 1.0) * 0.5
    ix0 = jnp.floor(ix)
    iy0 = jnp.floor(iy)
    wx1 = ix - ix0
    wx0 = 1.0 - wx1
    wy1 = iy - iy0
    wy0 = 1.0 - wy1
    vx0 = ((ix0 >= 0.0) & (ix0 <= 31.0)).astype(f32)
    vx1 = ((ix0 + 1.0 >= 0.0) & (ix0 + 1.0 <= 31.0)).astype(f32)
    vy0 = ((iy0 >= 0.0) & (iy0 <= 31.0)).astype(f32)
    vy1 = ((iy0 + 1.0 >= 0.0) & (iy0 + 1.0 <= 31.0)).astype(f32)
    x0 = jnp.clip(ix0, 0.0, 31.0).astype(jnp.int32)
    x1 = jnp.clip(ix0 + 1.0, 0.0, 31.0).astype(jnp.int32)
    y0 = jnp.clip(iy0, 0.0, 31.0).astype(jnp.int32)
    y1 = jnp.clip(iy0 + 1.0, 0.0, 31.0).astype(jnp.int32)

    idx_c = [y0 * 32 + x0, y0 * 32 + x1, y1 * 32 + x0, y1 * 32 + x1]
    wgt_c = [A * wx0 * wy0 * vx0 * vy0, A * wx1 * wy0 * vx1 * vy0,
             A * wx0 * wy1 * vx0 * vy1, A * wx1 * wy1 * vx1 * vy1]
    idx_out[...] = jnp.concatenate(
        [a[:, nh * NK:(nh + 1) * NK] for nh in range(NH) for a in idx_c],
        axis=1)
    wgt_out[...] = jnp.concatenate(
        [a[:, nh * NK:(nh + 1) * NK] for nh in range(NH) for a in wgt_c],
        axis=1)


def _run_proj(q2, k2, ref_rows, Wq, bq, Wk, bk, Woff_p, boff_p, WA, bA):
    R, C = q2.shape
    BR = 256
    grid = (R // BR,)
    nrb = ref_rows.shape[0] // BR
    row_spec = lambda n: pl.BlockSpec((BR, n), lambda i: (i, 0))
    full_spec = lambda a, b: pl.BlockSpec((a, b), lambda i: (0, 0))
    return pl.pallas_call(
        functools.partial(_proj_kernel, C // NH),
        grid=grid,
        in_specs=[
            row_spec(C), row_spec(C),
            pl.BlockSpec((BR, 2), lambda i: (i % nrb, 0)),
            full_spec(C, C), full_spec(C, C),
            full_spec(C, 2 * NH * NK), full_spec(C, NH * NK),
            full_spec(1, C), full_spec(1, C),
            full_spec(1, 2 * NH * NK), full_spec(1, NH * NK),
        ],
        out_specs=[pl.BlockSpec((BR * C // 128, 128), lambda i: (i, 0)),
                   row_spec(NJ * NH), row_spec(NJ * NH)],
        out_shape=[
            jax.ShapeDtypeStruct((R * C // 128, 128), jnp.float32),
            jax.ShapeDtypeStruct((R, NJ * NH), jnp.int32),
            jax.ShapeDtypeStruct((R, NJ * NH), jnp.float32),
        ],
    )(q2, k2, ref_rows, Wq, Wk, Woff_p, WA,
      bq.reshape(1, -1), bk.reshape(1, -1), boff_p.reshape(1, -1),
      bA.reshape(1, -1))


def _matmul_kernel(br, x_blk, W, b, out):
    x = x_blk[...].reshape(br, W.shape[0])
    out[...] = jnp.dot(x, W[...], preferred_element_type=jnp.float32) + b[...]


def _run_matmul(x12, W, b):
    C = W.shape[0]
    R = x12.shape[0] * 128 // C
    BR = 256
    return pl.pallas_call(
        functools.partial(_matmul_kernel, BR),
        grid=(R // BR,),
        in_specs=[
            pl.BlockSpec((BR * C // 128, 128), lambda i: (i, 0)),
            pl.BlockSpec((C, C), lambda i: (0, 0)),
            pl.BlockSpec((1, C), lambda i: (0, 0)),
        ],
        out_specs=pl.BlockSpec((BR, C), lambda i: (i, 0)),
        out_shape=jax.ShapeDtypeStruct((R, C), jnp.float32),
    )(x12, W, b.reshape(1, -1))


def _make_sample_kernel(BNH, HW, DK, interpret=False):
    HALF = HW // 2
    CHUNK = 64
    NSEG = DK // 16
    mesh = plsc.VectorSubcoreMesh(core_axis_name="c", subcore_axis_name="s",
                                  num_cores=2, num_subcores=16)

    @functools.partial(
        pl.kernel,
        mesh=mesh,
        out_type=jax.ShapeDtypeStruct((BNH, HW, DK), jnp.float32),
        scratch_types=[
            pltpu.VMEM((HW, DK), jnp.float32),
            pltpu.VMEM((HALF, NJ), jnp.int32),
            pltpu.VMEM((HALF, NJ), jnp.float32),
            pltpu.VMEM((2, CHUNK, DK), jnp.float32),
            pltpu.SemaphoreType.DMA,
            pltpu.SemaphoreType.DMA,
        ],
        compiler_params=pltpu.CompilerParams(use_tc_tiling_on_sc=False,
                                             needs_layout_passes=False),
        interpret=interpret,
    )
    def sample(sf_hbm, idx_hbm, wgt_hbm, out_hbm, table_v, idx_v, wgt_v,
               out_v, sem, osem):
        bh = lax.axis_index("s")
        half = lax.axis_index("c")
        b = bh // NH
        nh = bh % NH
        pbase = half * HALF
        rbase = b * HW + pbase
        copies = [
            pltpu.async_copy(
                sf_hbm.at[pl.ds(b * HW, HW), pl.ds(nh * DK, DK)], table_v,
                sem),
            pltpu.async_copy(
                idx_hbm.at[pl.ds(rbase, HALF), pl.ds(nh * NJ, NJ)], idx_v,
                sem),
            pltpu.async_copy(
                wgt_hbm.at[pl.ds(rbase, HALF), pl.ds(nh * NJ, NJ)], wgt_v,
                sem),
        ]
        for cp in copies:
            cp.wait()

        NCH = HALF // CHUNK

        def one_pos(p, buf, prow):
            iv = idx_v[p, :]
            wv = wgt_v[p, :]
            accs = None
            for j in range(NJ):
                row = iv[j]
                w = jnp.full((16,), wv[j])
                segs = [w * table_v[row, pl.ds(16 * c, 16)]
                        for c in range(NSEG)]
                accs = segs if accs is None else \
                    [a + s for a, s in zip(accs, segs)]
            for c in range(NSEG):
                out_v[buf, prow, pl.ds(16 * c, 16)] = accs[c]

        def chunk_body(ci, _):
            buf = lax.rem(ci, 2)

            @pl.when(ci >= 2)
            def _():
                pltpu.make_async_copy(
                    out_v.at[buf], out_hbm.at[bh, pl.ds(pbase, CHUNK), :],
                    osem).wait()

            def pair_body(q, _):
                prow = q * 2
                one_pos(ci * CHUNK + prow, buf, prow)
                one_pos(ci * CHUNK + prow + 1, buf, prow + 1)
                return 0

            lax.fori_loop(0, CHUNK // 2, pair_body, 0)
            pltpu.async_copy(
                out_v.at[buf],
                out_hbm.at[bh, pl.ds(pbase + ci * CHUNK, CHUNK), :], osem)
            return 0

        lax.fori_loop(0, NCH, chunk_body, 0)
        for t in range(min(2, NCH)):
            pltpu.make_async_copy(
                out_v.at[t], out_hbm.at[bh, pl.ds(pbase, CHUNK), :],
                osem).wait()

    return sample


def kernel(query, keys, ref_point, Wq, bq, Wk, bk, Woff, boff, WA, bA, Wm, bm):
    B, H, W, C = query.shape
    HW = H * W
    DK = C // NH
    f32 = jnp.float32

    q2 = query.reshape(B * HW, C)
    k2 = keys.reshape(B * HW, C)
    ref_rows = ref_point.reshape(HW, 2)

    cols = jnp.arange(NH * NK)
    perm = jnp.concatenate([(cols // NK) * (NK * 2) + (cols % NK) * 2,
                            (cols // NK) * (NK * 2) + (cols % NK) * 2 + 1])
    Woff_p = Woff[:, perm]
    boff_p = boff[perm]

    sf, idx, wgt = _run_proj(q2, k2, ref_rows, Wq, bq, Wk, bk,
                             Woff_p, boff_p, WA, bA)

    sf2 = sf.reshape(B * HW, C)
    feat = _make_sample_kernel(B * NH, HW, DK)(sf2, idx, wgt)

    out = _run_matmul(feat.reshape(B * HW * C // 128, 128), Wm, bm)
    return out.reshape(B, H, W, C)


# speedup vs baseline: 3.0600x; 1.0436x over previous
import functools

import jax
import jax.numpy as jnp
from jax import lax
from jax.experimental import pallas as pl
from jax.experimental.pallas import tpu as pltpu
from jax.experimental.pallas import tpu_sc as plsc

NH = 8
NK = 4
NCORNER = 4
NJ = NK * NCORNER


def _proj_kernel(dk, q_blk, k_blk, ref_blk, Wk, Woff, WA, bk, boff, bA,
                 sf_out, idx_out, wgt_out):
    f32 = jnp.float32
    sf = jnp.dot(k_blk[...], Wk[...], preferred_element_type=f32) + bk[...]
    sf_out[...] = sf.reshape(sf_out.shape)

    off = jnp.dot(q_blk[...], Woff[...], preferred_element_type=f32) + boff[...]
    logits = jnp.dot(q_blk[...], WA[...], preferred_element_type=f32) + bA[...]
    m = jnp.max(logits, axis=1, keepdims=True)
    e = jnp.exp(logits - m)
    gi = lax.broadcasted_iota(jnp.int32, (NH * NK, NH * NK), 0) // NK
    gj = lax.broadcasted_iota(jnp.int32, (NH * NK, NH * NK), 1) // NK
    G = (gi == gj).astype(f32)
    s = jnp.dot(e, G, preferred_element_type=f32)
    A = e / s

    offx = off[:, : NH * NK]
    offy = off[:, NH * NK:]
    rx = ref_blk[:, 0:1]
    ry = ref_blk[:, 1:2]
    px = rx * 31.0 + offx
    py = ry * 31.0 + offy
    ix = ((2.0 * px / 31.0) * 32.0 - 1.0) * 0.5
    iy = ((2.0 * py / 31.0) * 32.0 - 1.0) * 0.5
    ix0 = jnp.floor(ix)
    iy0 = jnp.floor(iy)
    wx1 = ix - ix0
    wx0 = 1.0 - wx1
    wy1 = iy - iy0
    wy0 = 1.0 - wy1
    vx0 = ((ix0 >= 0.0) & (ix0 <= 31.0)).astype(f32)
    vx1 = ((ix0 + 1.0 >= 0.0) & (ix0 + 1.0 <= 31.0)).astype(f32)
    vy0 = ((iy0 >= 0.0) & (iy0 <= 31.0)).astype(f32)
    vy1 = ((iy0 + 1.0 >= 0.0) & (iy0 + 1.0 <= 31.0)).astype(f32)
    x0 = jnp.clip(ix0, 0.0, 31.0).astype(jnp.int32)
    x1 = jnp.clip(ix0 + 1.0, 0.0, 31.0).astype(jnp.int32)
    y0 = jnp.clip(iy0, 0.0, 31.0).astype(jnp.int32)
    y1 = jnp.clip(iy0 + 1.0, 0.0, 31.0).astype(jnp.int32)

    idx_c = [y0 * 32 + x0, y0 * 32 + x1, y1 * 32 + x0, y1 * 32 + x1]
    wgt_c = [A * wx0 * wy0 * vx0 * vy0, A * wx1 * wy0 * vx1 * vy0,
             A * wx0 * wy1 * vx0 * vy1, A * wx1 * wy1 * vx1 * vy1]
    idx_out[...] = jnp.concatenate(
        [a[:, nh * NK:(nh + 1) * NK] for nh in range(NH) for a in idx_c],
        axis=1)
    wgt_out[...] = jnp.concatenate(
        [a[:, nh * NK:(nh + 1) * NK] for nh in range(NH) for a in wgt_c],
        axis=1)


def _run_proj(q2, k2, ref_rows, Wk, bk, Woff_p, boff_p, WA, bA):
    R, C = q2.shape
    BR = 256
    grid = (R // BR,)
    nrb = ref_rows.shape[0] // BR
    row_spec = lambda n: pl.BlockSpec((BR, n), lambda i: (i, 0))
    full_spec = lambda a, b: pl.BlockSpec((a, b), lambda i: (0, 0))
    return pl.pallas_call(
        functools.partial(_proj_kernel, C // NH),
        grid=grid,
        in_specs=[
            row_spec(C), row_spec(C),
            pl.BlockSpec((BR, 2), lambda i: (i % nrb, 0)),
            full_spec(C, C),
            full_spec(C, 2 * NH * NK), full_spec(C, NH * NK),
            full_spec(1, C),
            full_spec(1, 2 * NH * NK), full_spec(1, NH * NK),
        ],
        out_specs=[pl.BlockSpec((BR * C // 128, 128), lambda i: (i, 0)),
                   row_spec(NJ * NH), row_spec(NJ * NH)],
        out_shape=[
            jax.ShapeDtypeStruct((R * C // 128, 128), jnp.float32),
            jax.ShapeDtypeStruct((R, NJ * NH), jnp.int32),
            jax.ShapeDtypeStruct((R, NJ * NH), jnp.float32),
        ],
    )(q2, k2, ref_rows, Wk, Woff_p, WA,
      bk.reshape(1, -1), boff_p.reshape(1, -1), bA.reshape(1, -1))


def _matmul_kernel(br, x_blk, W, b, out):
    x = x_blk[...].reshape(br, W.shape[0])
    out[...] = jnp.dot(x, W[...], preferred_element_type=jnp.float32) + b[...]


def _run_matmul(x12, W, b):
    C = W.shape[0]
    R = x12.shape[0] * 128 // C
    BR = 256
    return pl.pallas_call(
        functools.partial(_matmul_kernel, BR),
        grid=(R // BR,),
        in_specs=[
            pl.BlockSpec((BR * C // 128, 128), lambda i: (i, 0)),
            pl.BlockSpec((C, C), lambda i: (0, 0)),
            pl.BlockSpec((1, C), lambda i: (0, 0)),
        ],
        out_specs=pl.BlockSpec((BR, C), lambda i: (i, 0)),
        out_shape=jax.ShapeDtypeStruct((R, C), jnp.float32),
    )(x12, W, b.reshape(1, -1))


def _make_sample_kernel(BNH, HW, DK, interpret=False):
    HALF = HW // 2
    CHUNK = 64
    NSEG = DK // 16
    mesh = plsc.VectorSubcoreMesh(core_axis_name="c", subcore_axis_name="s",
                                  num_cores=2, num_subcores=16)

    @functools.partial(
        pl.kernel,
        mesh=mesh,
        out_type=jax.ShapeDtypeStruct((BNH, HW, DK), jnp.float32),
        scratch_types=[
            pltpu.VMEM((HW, DK), jnp.float32),
            pltpu.VMEM((HALF, NJ), jnp.int32),
            pltpu.VMEM((HALF, NJ), jnp.float32),
            pltpu.VMEM((2, CHUNK, DK), jnp.float32),
            pltpu.SemaphoreType.DMA,
            pltpu.SemaphoreType.DMA,
        ],
        compiler_params=pltpu.CompilerParams(use_tc_tiling_on_sc=False,
                                             needs_layout_passes=False),
        interpret=interpret,
    )
    def sample(sf_hbm, idx_hbm, wgt_hbm, out_hbm, table_v, idx_v, wgt_v,
               out_v, sem, osem):
        bh = lax.axis_index("s")
        half = lax.axis_index("c")
        b = bh // NH
        nh = bh % NH
        pbase = half * HALF
        rbase = b * HW + pbase
        copies = [
            pltpu.async_copy(
                sf_hbm.at[pl.ds(b * HW, HW), pl.ds(nh * DK, DK)], table_v,
                sem),
            pltpu.async_copy(
                idx_hbm.at[pl.ds(rbase, HALF), pl.ds(nh * NJ, NJ)], idx_v,
                sem),
            pltpu.async_copy(
                wgt_hbm.at[pl.ds(rbase, HALF), pl.ds(nh * NJ, NJ)], wgt_v,
                sem),
        ]
        for cp in copies:
            cp.wait()

        NCH = HALF // CHUNK

        def one_pos(p, buf, prow):
            iv = idx_v[p, :]
            wv = wgt_v[p, :]
            accs = None
            for j in range(NJ):
                row = iv[j]
                w = jnp.full((16,), wv[j])
                segs = [w * table_v[row, pl.ds(16 * c, 16)]
                        for c in range(NSEG)]
                accs = segs if accs is None else \
                    [a + s for a, s in zip(accs, segs)]
            for c in range(NSEG):
                out_v[buf, prow, pl.ds(16 * c, 16)] = accs[c]

        def chunk_body(ci, _):
            buf = lax.rem(ci, 2)

            @pl.when(ci >= 2)
            def _():
                pltpu.make_async_copy(
                    out_v.at[buf], out_hbm.at[bh, pl.ds(pbase, CHUNK), :],
                    osem).wait()

            def pair_body(q, _):
                prow = q * 2
                one_pos(ci * CHUNK + prow, buf, prow)
                one_pos(ci * CHUNK + prow + 1, buf, prow + 1)
                return 0

            lax.fori_loop(0, CHUNK // 2, pair_body, 0)
            pltpu.async_copy(
                out_v.at[buf],
                out_hbm.at[bh, pl.ds(pbase + ci * CHUNK, CHUNK), :], osem)
            return 0

        lax.fori_loop(0, NCH, chunk_body, 0)
        for t in range(min(2, NCH)):
            pltpu.make_async_copy(
                out_v.at[t], out_hbm.at[bh, pl.ds(pbase, CHUNK), :],
                osem).wait()

    return sample


def kernel(query, keys, ref_point, Wq, bq, Wk, bk, Woff, boff, WA, bA, Wm, bm):
    B, H, W, C = query.shape
    HW = H * W
    DK = C // NH
    f32 = jnp.float32

    q2 = query.reshape(B * HW, C)
    k2 = keys.reshape(B * HW, C)
    ref_rows = ref_point.reshape(HW, 2)

    cols = jnp.arange(NH * NK)
    perm = jnp.concatenate([(cols // NK) * (NK * 2) + (cols % NK) * 2,
                            (cols // NK) * (NK * 2) + (cols % NK) * 2 + 1])
    Woff_p = jnp.dot(Wq, Woff, preferred_element_type=f32)[:, perm]
    boff_p = (jnp.dot(bq, Woff, preferred_element_type=f32) + boff)[perm]
    WA_f = jnp.dot(Wq, WA, preferred_element_type=f32)
    bA_f = jnp.dot(bq, WA, preferred_element_type=f32) + bA

    sf, idx, wgt = _run_proj(q2, k2, ref_rows, Wk, bk,
                             Woff_p, boff_p, WA_f, bA_f)

    sf2 = sf.reshape(B * HW, C)
    feat = _make_sample_kernel(B * NH, HW, DK)(sf2, idx, wgt)

    out = _run_matmul(feat.reshape(B * HW * C // 128, 128), Wm, bm)
    return out.reshape(B, H, W, C)


# speedup vs baseline: 3.6197x; 1.1829x over previous
import functools

import jax
import jax.numpy as jnp
from jax import lax
from jax.experimental import pallas as pl
from jax.experimental.pallas import tpu as pltpu
from jax.experimental.pallas import tpu_sc as plsc

NH = 8
NK = 4
NCORNER = 4
NJ = NK * NCORNER


def _proj_kernel(dk, q_blk, k_blk, ref_blk, Wk, Woff, WA, bk, boff, bA,
                 sf_out, idx_out, wgt_out):
    f32 = jnp.float32
    sf = jnp.dot(k_blk[...], Wk[...], preferred_element_type=f32) + bk[...]
    sf_out[...] = sf.reshape(sf_out.shape)

    off = jnp.dot(q_blk[...], Woff[...], preferred_element_type=f32) + boff[...]
    logits = jnp.dot(q_blk[...], WA[...], preferred_element_type=f32) + bA[...]
    m = jnp.max(logits, axis=1, keepdims=True)
    e = jnp.exp(logits - m)
    gi = lax.broadcasted_iota(jnp.int32, (NH * NK, NH * NK), 0) // NK
    gj = lax.broadcasted_iota(jnp.int32, (NH * NK, NH * NK), 1) // NK
    G = (gi == gj).astype(f32)
    s = jnp.dot(e, G, preferred_element_type=f32)
    A = e / s

    offx = off[:, : NH * NK]
    offy = off[:, NH * NK:]
    rx = ref_blk[:, 0:1]
    ry = ref_blk[:, 1:2]
    px = rx * 31.0 + offx
    py = ry * 31.0 + offy
    ix = ((2.0 * px / 31.0) * 32.0 - 1.0) * 0.5
    iy = ((2.0 * py / 31.0) * 32.0 - 1.0) * 0.5
    ix0 = jnp.floor(ix)
    iy0 = jnp.floor(iy)
    wx1 = ix - ix0
    wx0 = 1.0 - wx1
    wy1 = iy - iy0
    wy0 = 1.0 - wy1
    vx0 = ((ix0 >= 0.0) & (ix0 <= 31.0)).astype(f32)
    vx1 = ((ix0 + 1.0 >= 0.0) & (ix0 + 1.0 <= 31.0)).astype(f32)
    vy0 = ((iy0 >= 0.0) & (iy0 <= 31.0)).astype(f32)
    vy1 = ((iy0 + 1.0 >= 0.0) & (iy0 + 1.0 <= 31.0)).astype(f32)
    x0 = jnp.clip(ix0, 0.0, 31.0)
    x1 = jnp.clip(ix0 + 1.0, 0.0, 31.0)
    y0 = jnp.clip(iy0, 0.0, 31.0)
    y1 = jnp.clip(iy0 + 1.0, 0.0, 31.0)

    x_cm = jnp.concatenate([x0, x1, x0, x1], axis=1)
    y_cm = jnp.concatenate([y0, y0, y1, y1], axis=1)
    wgt_cm = jnp.concatenate(
        [A * wx0 * wy0 * vx0 * vy0, A * wx1 * wy0 * vx1 * vy0,
         A * wx0 * wy1 * vx0 * vy1, A * wx1 * wy1 * vx1 * vy1], axis=1)
    NC = NCORNER * NH * NK
    si = lax.broadcasted_iota(jnp.int32, (NC, NC), 0)
    tj = lax.broadcasted_iota(jnp.int32, (NC, NC), 1)
    P = (tj == ((si % 32) // NK) * 16 + (si // 32) * NK + si % NK).astype(f32)
    xP = jnp.dot(x_cm, P, preferred_element_type=f32)
    yP = jnp.dot(y_cm, P, preferred_element_type=f32)
    idx_out[...] = (yP * 32.0 + xP + 0.5).astype(jnp.int32)
    wgt_out[...] = jnp.dot(wgt_cm, P, preferred_element_type=f32)


def _run_proj(q2, k2, ref_rows, Wk, bk, Woff_p, boff_p, WA, bA):
    R, C = q2.shape
    BR = 256
    grid = (R // BR,)
    nrb = ref_rows.shape[0] // BR
    row_spec = lambda n: pl.BlockSpec((BR, n), lambda i: (i, 0))
    full_spec = lambda a, b: pl.BlockSpec((a, b), lambda i: (0, 0))
    return pl.pallas_call(
        functools.partial(_proj_kernel, C // NH),
        grid=grid,
        in_specs=[
            row_spec(C), row_spec(C),
            pl.BlockSpec((BR, 2), lambda i: (i % nrb, 0)),
            full_spec(C, C),
            full_spec(C, 2 * NH * NK), full_spec(C, NH * NK),
            full_spec(1, C),
            full_spec(1, 2 * NH * NK), full_spec(1, NH * NK),
        ],
        out_specs=[pl.BlockSpec((BR * C // 128, 128), lambda i: (i, 0)),
                   row_spec(NJ * NH), row_spec(NJ * NH)],
        out_shape=[
            jax.ShapeDtypeStruct((R * C // 128, 128), jnp.float32),
            jax.ShapeDtypeStruct((R, NJ * NH), jnp.int32),
            jax.ShapeDtypeStruct((R, NJ * NH), jnp.float32),
        ],
    )(q2, k2, ref_rows, Wk, Woff_p, WA,
      bk.reshape(1, -1), boff_p.reshape(1, -1), bA.reshape(1, -1))


def _matmul_kernel(br, x_blk, W, b, out):
    x = x_blk[...].reshape(br, W.shape[0])
    out[...] = jnp.dot(x, W[...], preferred_element_type=jnp.float32) + b[...]


def _run_matmul(x12, W, b):
    C = W.shape[0]
    R = x12.shape[0] * 128 // C
    BR = 256
    return pl.pallas_call(
        functools.partial(_matmul_kernel, BR),
        grid=(R // BR,),
        in_specs=[
            pl.BlockSpec((BR * C // 128, 128), lambda i: (i, 0)),
            pl.BlockSpec((C, C), lambda i: (0, 0)),
            pl.BlockSpec((1, C), lambda i: (0, 0)),
        ],
        out_specs=pl.BlockSpec((BR, C), lambda i: (i, 0)),
        out_shape=jax.ShapeDtypeStruct((R, C), jnp.float32),
    )(x12, W, b.reshape(1, -1))


def _make_sample_kernel(BNH, HW, DK, interpret=False):
    HALF = HW // 2
    CHUNK = 64
    NSEG = DK // 16
    mesh = plsc.VectorSubcoreMesh(core_axis_name="c", subcore_axis_name="s",
                                  num_cores=2, num_subcores=16)

    @functools.partial(
        pl.kernel,
        mesh=mesh,
        out_type=jax.ShapeDtypeStruct((BNH, HW, DK), jnp.float32),
        scratch_types=[
            pltpu.VMEM((HW, DK), jnp.float32),
            pltpu.VMEM((HALF, NJ), jnp.int32),
            pltpu.VMEM((HALF, NJ), jnp.float32),
            pltpu.VMEM((2, CHUNK, DK), jnp.float32),
            pltpu.SemaphoreType.DMA,
            pltpu.SemaphoreType.DMA,
        ],
        compiler_params=pltpu.CompilerParams(use_tc_tiling_on_sc=False,
                                             needs_layout_passes=False),
        interpret=interpret,
    )
    def sample(sf_hbm, idx_hbm, wgt_hbm, out_hbm, table_v, idx_v, wgt_v,
               out_v, sem, osem):
        bh = lax.axis_index("s")
        half = lax.axis_index("c")
        b = bh // NH
        nh = bh % NH
        pbase = half * HALF
        rbase = b * HW + pbase
        copies = [
            pltpu.async_copy(
                sf_hbm.at[pl.ds(b * HW, HW), pl.ds(nh * DK, DK)], table_v,
                sem),
            pltpu.async_copy(
                idx_hbm.at[pl.ds(rbase, HALF), pl.ds(nh * NJ, NJ)], idx_v,
                sem),
            pltpu.async_copy(
                wgt_hbm.at[pl.ds(rbase, HALF), pl.ds(nh * NJ, NJ)], wgt_v,
                sem),
        ]
        for cp in copies:
            cp.wait()

        NCH = HALF // CHUNK

        def one_pos(p, buf, prow):
            iv = idx_v[p, :]
            wv = wgt_v[p, :]
            accs = None
            for j in range(NJ):
                row = iv[j]
                w = jnp.full((16,), wv[j])
                segs = [w * table_v[row, pl.ds(16 * c, 16)]
                        for c in range(NSEG)]
                accs = segs if accs is None else \
                    [a + s for a, s in zip(accs, segs)]
            for c in range(NSEG):
                out_v[buf, prow, pl.ds(16 * c, 16)] = accs[c]

        def chunk_body(ci, _):
            buf = lax.rem(ci, 2)

            @pl.when(ci >= 2)
            def _():
                pltpu.make_async_copy(
                    out_v.at[buf], out_hbm.at[bh, pl.ds(pbase, CHUNK), :],
                    osem).wait()

            def pair_body(q, _):
                prow = q * 2
                one_pos(ci * CHUNK + prow, buf, prow)
                one_pos(ci * CHUNK + prow + 1, buf, prow + 1)
                return 0

            lax.fori_loop(0, CHUNK // 2, pair_body, 0)
            pltpu.async_copy(
                out_v.at[buf],
                out_hbm.at[bh, pl.ds(pbase + ci * CHUNK, CHUNK), :], osem)
            return 0

        lax.fori_loop(0, NCH, chunk_body, 0)
        for t in range(min(2, NCH)):
            pltpu.make_async_copy(
                out_v.at[t], out_hbm.at[bh, pl.ds(pbase, CHUNK), :],
                osem).wait()

    return sample


def kernel(query, keys, ref_point, Wq, bq, Wk, bk, Woff, boff, WA, bA, Wm, bm):
    B, H, W, C = query.shape
    HW = H * W
    DK = C // NH
    f32 = jnp.float32

    q2 = query.reshape(B * HW, C)
    k2 = keys.reshape(B * HW, C)
    ref_rows = ref_point.reshape(HW, 2)

    cols = jnp.arange(NH * NK)
    perm = jnp.concatenate([(cols // NK) * (NK * 2) + (cols % NK) * 2,
                            (cols // NK) * (NK * 2) + (cols % NK) * 2 + 1])
    Woff_p = jnp.dot(Wq, Woff, preferred_element_type=f32)[:, perm]
    boff_p = (jnp.dot(bq, Woff, preferred_element_type=f32) + boff)[perm]
    WA_f = jnp.dot(Wq, WA, preferred_element_type=f32)
    bA_f = jnp.dot(bq, WA, preferred_element_type=f32) + bA

    sf, idx, wgt = _run_proj(q2, k2, ref_rows, Wk, bk,
                             Woff_p, boff_p, WA_f, bA_f)

    sf2 = sf.reshape(B * HW, C)
    feat = _make_sample_kernel(B * NH, HW, DK)(sf2, idx, wgt)

    out = _run_matmul(feat.reshape(B * HW * C // 128, 128), Wm, bm)
    return out.reshape(B, H, W, C)
